# Initial kernel scaffold; baseline (speedup 1.0000x reference)
import jax
import jax.numpy as jnp
from jax.experimental import pallas as pl


def kernel(rel0, rel1, rel2, rel3, node_init, relW1_0, relb1_0, relW2_0, relb2_0, relW1_1, relb1_1, relW2_1, relb2_1, relW1_2, relb1_2, relW2_2, relb2_2, relW1_3, relb1_3, relW2_3, relb2_3, updW1, updb1, updW2, updb2, preW1, preb1, preW2, preb2, postW1, postb1, postW2, postb2, ruW1, rub1, ruW2, rub2):
    raise NotImplementedError("write your pallas kernel here")


# speedup vs baseline: 1.7933x; 1.7933x over previous
import functools

import jax
import jax.numpy as jnp
from jax import lax
from jax.experimental import pallas as pl
from jax.experimental.pallas import tpu as pltpu
from jax.experimental.pallas import tpu_sc as plsc

H = 128
N = 10000
NPAD = 10240
ARITIES = (1, 2, 2, 3)
NT = (10000, 160000, 160000, 50000)
TPAD = (10112, 160000, 160000, 50048)
EPAD = tuple(t * a for t, a in zip(TPAD, ARITIES))
EOFF = (0, 10112, 330112, 650112)
EALL = 800256
CHUNK = 128
NCHUNKS = EALL // CHUNK
NCK = tuple(e // CHUNK for e in EPAD)
COFF = tuple(o // CHUNK for o in EOFF)
NW = 32
SEG = 625
B = 16
F32 = jnp.float32

_mesh = plsc.VectorSubcoreMesh(core_axis_name="c", subcore_axis_name="s")


@functools.partial(
    pl.kernel,
    out_type=jax.ShapeDtypeStruct((EALL, H), F32),
    mesh=_mesh,
    scratch_types=[
        pltpu.VMEM((2, CHUNK), jnp.int32),
        pltpu.VMEM((CHUNK, H), F32),
        pltpu.SemaphoreType.DMA,
    ],
)
def _sc_gather(idx_hbm, tab_hbm, out_hbm, idx_v, rows_v, sem):
    c = lax.axis_index("c")
    s = lax.axis_index("s")
    w = s * 2 + c

    def body(j, carry):
        cid = w + j * NW

        @pl.when(cid < NCHUNKS)
        def _():
            pltpu.sync_copy(idx_hbm.at[cid], idx_v.at[0])
            pltpu.async_copy(tab_hbm.at[idx_v.at[0]], rows_v, sem).wait()
            pltpu.sync_copy(rows_v, out_hbm.at[pl.ds(cid * CHUNK, CHUNK)])

        return carry

    lax.fori_loop(0, (NCHUNKS + NW - 1) // NW, body, 0)


@functools.partial(
    pl.kernel,
    out_type=jax.ShapeDtypeStruct((2, NPAD, H), F32),
    mesh=_mesh,
    scratch_types=[
        pltpu.VMEM((2, CHUNK), jnp.int32),
        pltpu.VMEM((CHUNK, H), F32),
        pltpu.VMEM((16,), F32),
        pltpu.VMEM_SHARED((NPAD, H), F32),
        pltpu.SemaphoreType.DMA,
    ],
)
def _sc_scatter(idx_hbm, o0, o1, o2, o3, m_hbm, zer_hbm, out_hbm,
                idx_v, rows_v, m_v, accum, sem):
    c = lax.axis_index("c")
    s = lax.axis_index("s")
    w = s * 2 + c
    slab = NPAD // 16

    pltpu.sync_copy(zer_hbm, accum.at[pl.ds(s * slab, slab)])
    pltpu.sync_copy(m_hbm, m_v)
    plsc.subcore_barrier()
    mvec = m_v[...]

    for r in range(4):
        o_r = (o0, o1, o2, o3)[r]
        nck = NCK[r]
        coff = COFF[r]

        def body(j, carry, o_r=o_r, nck=nck, coff=coff):
            lc = w + j * NW

            @pl.when(lc < nck)
            def _():
                pltpu.sync_copy(idx_hbm.at[coff + lc], idx_v.at[0])
                pltpu.sync_copy(o_r.at[pl.ds(lc * CHUNK, CHUNK)], rows_v)

                def erow(i, cc):
                    for jj in range(H // 16):
                        sl = pl.ds(jj * 16, 16)
                        rows_v[i, sl] = jnp.exp((rows_v[i, sl] - mvec) * 8.0)
                    return cc

                lax.fori_loop(0, CHUNK, erow, 0)
                pltpu.sync_copy(rows_v, accum.at[idx_v.at[0]], add=True)

            return carry

        lax.fori_loop(0, (nck + NW - 1) // NW, body, 0)

    plsc.subcore_barrier()
    pltpu.sync_copy(accum.at[pl.ds(s * slab, slab)],
                    out_hbm.at[c, pl.ds(s * slab, slab)])


def _rel_mlp(x, w1, b1, w2, b2, t_valid, tm):
    tpad, d = x.shape
    grid = tpad // tm

    def kern(x_ref, w1_ref, b1_ref, w2_ref, b2_ref, o_ref, mx_ref):
        i = pl.program_id(0)
        xv = x_ref[...]
        h = jnp.maximum(
            jnp.dot(xv, w1_ref[...], preferred_element_type=F32) + b1_ref[...],
            0.0)
        o = jnp.dot(h, w2_ref[...], preferred_element_type=F32) + b2_ref[...]
        o_ref[...] = o
        row = i * tm + lax.broadcasted_iota(jnp.int32, (tm, 1), 0)
        om = jnp.where(row < t_valid, o, -jnp.inf)
        m = jnp.max(om)

        @pl.when(i == 0)
        def _():
            mx_ref[0, 0] = m

        @pl.when(i > 0)
        def _():
            mx_ref[0, 0] = jnp.maximum(mx_ref[0, 0], m)

    return pl.pallas_call(
        kern,
        grid=(grid,),
        in_specs=[
            pl.BlockSpec((tm, d), lambda i: (i, 0)),
            pl.BlockSpec((d, d), lambda i: (0, 0)),
            pl.BlockSpec((1, d), lambda i: (0, 0)),
            pl.BlockSpec((d, d), lambda i: (0, 0)),
            pl.BlockSpec((1, d), lambda i: (0, 0)),
        ],
        out_specs=[
            pl.BlockSpec((tm, d), lambda i: (i, 0)),
            pl.BlockSpec(memory_space=pltpu.SMEM),
        ],
        out_shape=[
            jax.ShapeDtypeStruct((tpad, d), F32),
            jax.ShapeDtypeStruct((1, 1), F32),
        ],
    )(x, w1, b1, w2, b2)


def _update1(S, ns, m, uw1, ub1, uw2, ub2, pw1, pb1, pw2, pb2):
    tm = 2560
    grid = NPAD // tm

    def kern(s_ref, ns_ref, m_ref, uw1_ref, ub1_ref, uw2_ref, ub2_ref,
             pw1_ref, pb1_ref, pw2_ref, pb2_ref, ns2_ref, agg_ref):
        i = pl.program_id(0)
        sv = s_ref[0] + s_ref[1]
        row = i * tm + lax.broadcasted_iota(jnp.int32, (tm, 1), 0)
        mm = jnp.where(row < N,
                       jnp.log(sv + 1e-16) * 0.125 + m_ref[0, 0], 0.0)
        u = jnp.concatenate([mm, ns_ref[...]], axis=1)
        h = jnp.maximum(
            jnp.dot(u, uw1_ref[...], preferred_element_type=F32)
            + ub1_ref[...], 0.0)
        ns2 = jnp.dot(h, uw2_ref[...], preferred_element_type=F32) + ub2_ref[...]
        ns2_ref[...] = ns2
        hp = jnp.maximum(
            jnp.dot(ns2, pw1_ref[...], preferred_element_type=F32)
            + pb1_ref[...], 0.0)
        pre = jnp.dot(hp, pw2_ref[...], preferred_element_type=F32) + pb2_ref[...]
        rowt = i * tm + lax.broadcasted_iota(jnp.int32, (1, tm), 1)
        segt = rowt // SEG
        oh = jnp.where(
            (segt == lax.broadcasted_iota(jnp.int32, (B, tm), 0))
            & (rowt < N), 1.0, 0.0).astype(F32)
        part = jnp.dot(oh, pre, preferred_element_type=F32)

        @pl.when(i == 0)
        def _():
            agg_ref[...] = part

        @pl.when(i > 0)
        def _():
            agg_ref[...] = agg_ref[...] + part

    return pl.pallas_call(
        kern,
        grid=(grid,),
        in_specs=[
            pl.BlockSpec((2, tm, H), lambda i: (0, i, 0)),
            pl.BlockSpec((tm, H), lambda i: (i, 0)),
            pl.BlockSpec(memory_space=pltpu.SMEM),
            pl.BlockSpec((2 * H, 2 * H), lambda i: (0, 0)),
            pl.BlockSpec((1, 2 * H), lambda i: (0, 0)),
            pl.BlockSpec((2 * H, H), lambda i: (0, 0)),
            pl.BlockSpec((1, H), lambda i: (0, 0)),
            pl.BlockSpec((H, H), lambda i: (0, 0)),
            pl.BlockSpec((1, H), lambda i: (0, 0)),
            pl.BlockSpec((H, H), lambda i: (0, 0)),
            pl.BlockSpec((1, H), lambda i: (0, 0)),
        ],
        out_specs=[
            pl.BlockSpec((tm, H), lambda i: (i, 0)),
            pl.BlockSpec((B, H), lambda i: (0, 0)),
        ],
        out_shape=[
            jax.ShapeDtypeStruct((NPAD, H), F32),
            jax.ShapeDtypeStruct((B, H), F32),
        ],
    )(S, ns, m, uw1, ub1, uw2, ub2, pw1, pb1, pw2, pb2)


def _update2(ns2, agg, ow1, ob1, ow2, ob2, rw1, rb1, rw2, rb2):
    tm = 2560
    grid = NPAD // tm

    def kern(ns2_ref, agg_ref, ow1_ref, ob1_ref, ow2_ref, ob2_ref,
             rw1_ref, rb1_ref, rw2_ref, rb2_ref, out_ref):
        i = pl.program_id(0)
        hg = jnp.maximum(
            jnp.dot(agg_ref[...], ow1_ref[...], preferred_element_type=F32)
            + ob1_ref[...], 0.0)
        post = jnp.dot(hg, ow2_ref[...], preferred_element_type=F32) + ob2_ref[...]
        rowt = i * tm + lax.broadcasted_iota(jnp.int32, (1, tm), 1)
        segt = rowt // SEG
        oh = jnp.where(
            (segt == lax.broadcasted_iota(jnp.int32, (B, tm), 0))
            & (rowt < N), 1.0, 0.0).astype(F32)
        rmsg = lax.dot_general(oh, post, (((0,), (0,)), ((), ())),
                               preferred_element_type=F32)
        ru = jnp.concatenate([ns2_ref[...], rmsg], axis=1)
        h = jnp.maximum(
            jnp.dot(ru, rw1_ref[...], preferred_element_type=F32)
            + rb1_ref[...], 0.0)
        out = jnp.dot(h, rw2_ref[...], preferred_element_type=F32) + rb2_ref[...]
        row = i * tm + lax.broadcasted_iota(jnp.int32, (tm, 1), 0)
        out_ref[...] = jnp.where(row < N, out, 0.0)

    return pl.pallas_call(
        kern,
        grid=(grid,),
        in_specs=[
            pl.BlockSpec((tm, H), lambda i: (i, 0)),
            pl.BlockSpec((B, H), lambda i: (0, 0)),
            pl.BlockSpec((H, H), lambda i: (0, 0)),
            pl.BlockSpec((1, H), lambda i: (0, 0)),
            pl.BlockSpec((H, H), lambda i: (0, 0)),
            pl.BlockSpec((1, H), lambda i: (0, 0)),
            pl.BlockSpec((2 * H, 2 * H), lambda i: (0, 0)),
            pl.BlockSpec((1, 2 * H), lambda i: (0, 0)),
            pl.BlockSpec((2 * H, H), lambda i: (0, 0)),
            pl.BlockSpec((1, H), lambda i: (0, 0)),
        ],
        out_specs=pl.BlockSpec((tm, H), lambda i: (i, 0)),
        out_shape=jax.ShapeDtypeStruct((NPAD, H), F32),
    )(ns2, agg, ow1, ob1, ow2, ob2, rw1, rb1, rw2, rb2)


def kernel(rel0, rel1, rel2, rel3, node_init,
           relW1_0, relb1_0, relW2_0, relb2_0,
           relW1_1, relb1_1, relW2_1, relb2_1,
           relW1_2, relb1_2, relW2_2, relb2_2,
           relW1_3, relb1_3, relW2_3, relb2_3,
           updW1, updb1, updW2, updb2,
           preW1, preb1, preW2, preb2,
           postW1, postb1, postW2, postb2,
           ruW1, rub1, ruW2, rub2):
    rels = (rel0, rel1, rel2, rel3)
    relW = ((relW1_0, relb1_0, relW2_0, relb2_0),
            (relW1_1, relb1_1, relW2_1, relb2_1),
            (relW1_2, relb1_2, relW2_2, relb2_2),
            (relW1_3, relb1_3, relW2_3, relb2_3))
    tms = (1264, 2000, 2000, 2176)

    idx_parts = []
    for r in range(4):
        v = rels[r]
        pad = EPAD[r] - v.shape[0]
        idx_parts.append(jnp.pad(v, (0, pad), constant_values=N))
    idx2 = jnp.concatenate(idx_parts).reshape(NCHUNKS, CHUNK)

    ns = jnp.pad(node_init, ((0, NPAD - N), (0, 0)))
    zer = jnp.zeros((NPAD // 16, H), F32)

    b2d = lambda b: b.reshape(1, -1)

    for _ in range(3):
        x_all = _sc_gather(idx2, ns)
        os, ms = [], []
        for r in range(4):
            a = ARITIES[r]
            xr = x_all[EOFF[r]:EOFF[r] + EPAD[r]].reshape(TPAD[r], a * H)
            w1, b1, w2, b2 = relW[r]
            o, m = _rel_mlp(xr, w1, b2d(b1), w2, b2d(b2), NT[r], tms[r])
            os.append(o.reshape(EPAD[r], H))
            ms.append(m)
        mx = jnp.maximum(jnp.maximum(ms[0], ms[1]),
                         jnp.maximum(ms[2], ms[3]))
        m16 = jnp.broadcast_to(mx.reshape(1), (16,))
        S = _sc_scatter(idx2, os[0], os[1], os[2], os[3], m16, zer)
        ns2, agg = _update1(S, ns, mx, updW1, b2d(updb1), updW2, b2d(updb2),
                            preW1, b2d(preb1), preW2, b2d(preb2))
        ns = _update2(ns2, agg, postW1, b2d(postb1), postW2, b2d(postb2),
                      ruW1, b2d(rub1), ruW2, b2d(rub2))

    return ns[:N]
